# baseline (speedup 1.0000x reference)
import functools
import math

import jax
import jax.numpy as jnp
from jax import lax
from jax.experimental import pallas as pl
from jax.experimental.pallas import tpu as pltpu
from jax.experimental.pallas import tpu_sc as plsc

PAD = 1
LANES = 16
NUM_CORES = 2
NUM_SUBCORES = 16
NUM_WORKERS = NUM_CORES * NUM_SUBCORES


def _build_positions_sc(n_tok):
  per_w = n_tok // NUM_WORKERS
  n_vregs = per_w // LANES
  w_per_row = 8192 // per_w

  mesh = plsc.VectorSubcoreMesh(
      core_axis_name="c", subcore_axis_name="s", num_cores=NUM_CORES,
      num_subcores=NUM_SUBCORES)

  @functools.partial(
      pl.kernel,
      mesh=mesh,
      compiler_params=pltpu.CompilerParams(needs_layout_passes=False),
      out_type=jax.ShapeDtypeStruct((n_tok,), jnp.int32),
      scratch_types=[
          pltpu.VMEM((per_w,), jnp.int32),
          pltpu.VMEM((per_w,), jnp.int32),
          pltpu.VMEM((LANES,), jnp.int32),
          pltpu.VMEM((NUM_SUBCORES * LANES,), jnp.int32),
          pltpu.VMEM_SHARED((NUM_SUBCORES * LANES,), jnp.int32),
      ],
  )
  def k(ids_hbm, pos_hbm, ids_v, pos_v, stage_v, tot_v, tot_sh):
    cid = lax.axis_index("c")
    sid = lax.axis_index("s")
    wid = cid * NUM_SUBCORES + sid
    base = wid * per_w

    pltpu.sync_copy(ids_hbm.at[pl.ds(base, per_w)], ids_v)

    def body(i, carry):
      ids = ids_v[pl.ds(i * LANES, LANES)]
      m = jnp.minimum(jnp.abs(ids - PAD), 1)
      c = plsc.cumsum(m)
      pos_v[pl.ds(i * LANES, LANES)] = c + carry
      return carry + jnp.sum(m)

    total = lax.fori_loop(0, n_vregs, body, jnp.int32(0))

    stage_v[...] = jnp.full((LANES,), total, jnp.int32)
    pltpu.sync_copy(stage_v, tot_sh.at[pl.ds(sid * LANES, LANES)])
    plsc.subcore_barrier()

    pltpu.sync_copy(tot_sh, tot_v)
    r0 = (sid // w_per_row) * w_per_row
    offset = lax.fori_loop(
        r0, sid, lambda j, acc: acc + tot_v[pl.ds(j * LANES, LANES)],
        jnp.zeros((LANES,), jnp.int32))

    def body2(i, carry):
      ids = ids_v[pl.ds(i * LANES, LANES)]
      m = jnp.minimum(jnp.abs(ids - PAD), 1)
      c = pos_v[pl.ds(i * LANES, LANES)]
      pos_v[pl.ds(i * LANES, LANES)] = (c + offset) * m + PAD
      return carry

    lax.fori_loop(0, n_vregs, body2, 0)
    pltpu.sync_copy(pos_v, pos_hbm.at[pl.ds(base, per_w)])

  return k


def _build_rows_tc(n_tok, dim, blk):
  half = dim // 2

  def body(pos_ref, freq_ref, out_ref):
    p = pos_ref[...]
    f = freq_ref[...]
    emb = p * f
    nonpad = (p != float(PAD)).astype(jnp.float32)
    out_ref[...] = jnp.concatenate(
        [jnp.sin(emb), jnp.cos(emb)], axis=1) * nonpad

  return pl.pallas_call(
      body,
      grid=(n_tok // blk,),
      in_specs=[
          pl.BlockSpec((blk, 1), lambda i: (i, 0)),
          pl.BlockSpec((1, half), lambda i: (0, 0)),
      ],
      out_specs=pl.BlockSpec((blk, dim), lambda i: (i, 0)),
      out_shape=jax.ShapeDtypeStruct((n_tok, dim), jnp.float32),
  )


def kernel(input, weights):
  bsz, seq_len = input.shape
  dim = weights.shape[1]
  n_tok = bsz * seq_len
  half = dim // 2

  pos = _build_positions_sc(n_tok)(input.reshape(-1))
  pos_f = pos.astype(jnp.float32).reshape(n_tok, 1)
  scale = math.log(10000.0) / (half - 1)
  freq = jnp.exp(jnp.arange(half, dtype=jnp.float32) * -scale).reshape(1, half)

  out = _build_rows_tc(n_tok, dim, 256)(pos_f, freq)
  return out.reshape(bsz, seq_len, dim)


# speedup vs baseline: 1.9804x; 1.9804x over previous
import functools
import math

import jax
import jax.numpy as jnp
from jax import lax
from jax.experimental import pallas as pl
from jax.experimental.pallas import tpu as pltpu
from jax.experimental.pallas import tpu_sc as plsc

PAD = 1
LANES = 16
NUM_CORES = 2
NUM_SUBCORES = 16
NUM_WORKERS = NUM_CORES * NUM_SUBCORES


def _build_positions_sc(n_tok):
  per_w = n_tok // NUM_WORKERS
  n_vregs = per_w // LANES
  w_per_row = 8192 // per_w

  mesh = plsc.VectorSubcoreMesh(
      core_axis_name="c", subcore_axis_name="s", num_cores=NUM_CORES,
      num_subcores=NUM_SUBCORES)

  @functools.partial(
      pl.kernel,
      mesh=mesh,
      compiler_params=pltpu.CompilerParams(needs_layout_passes=False),
      out_type=jax.ShapeDtypeStruct((n_tok,), jnp.int32),
      scratch_types=[
          pltpu.VMEM((per_w,), jnp.int32),
          pltpu.VMEM((per_w,), jnp.int32),
          pltpu.VMEM((LANES,), jnp.int32),
          pltpu.VMEM((NUM_SUBCORES * LANES,), jnp.int32),
          pltpu.VMEM_SHARED((NUM_SUBCORES * LANES,), jnp.int32),
      ],
  )
  def k(ids_hbm, pos_hbm, ids_v, pos_v, stage_v, tot_v, tot_sh):
    cid = lax.axis_index("c")
    sid = lax.axis_index("s")
    wid = cid * NUM_SUBCORES + sid
    base = wid * per_w

    pltpu.sync_copy(ids_hbm.at[pl.ds(base, per_w)], ids_v)

    def body(i, carry):
      ids = ids_v[pl.ds(i * LANES, LANES)]
      m = jnp.minimum(jnp.abs(ids - PAD), 1)
      c = plsc.cumsum(m)
      pos_v[pl.ds(i * LANES, LANES)] = c + carry
      return carry + jnp.sum(m)

    total = lax.fori_loop(0, n_vregs, body, jnp.int32(0))

    stage_v[...] = jnp.full((LANES,), total, jnp.int32)
    pltpu.sync_copy(stage_v, tot_sh.at[pl.ds(sid * LANES, LANES)])
    plsc.subcore_barrier()

    pltpu.sync_copy(tot_sh, tot_v)
    r0 = (sid // w_per_row) * w_per_row
    offset = lax.fori_loop(
        r0, sid, lambda j, acc: acc + tot_v[pl.ds(j * LANES, LANES)],
        jnp.zeros((LANES,), jnp.int32))

    def body2(i, carry):
      ids = ids_v[pl.ds(i * LANES, LANES)]
      m = jnp.minimum(jnp.abs(ids - PAD), 1)
      c = pos_v[pl.ds(i * LANES, LANES)]
      pos_v[pl.ds(i * LANES, LANES)] = (c + offset) * m + PAD
      return carry

    lax.fori_loop(0, n_vregs, body2, 0)
    pltpu.sync_copy(pos_v, pos_hbm.at[pl.ds(base, per_w)])

  return k


SPLIT = 64
NA = 136
NB = 64


def _build_rows_tc(n_tok, dim, blk):
  half = dim // 2

  def body(pos_ref, sa_ref, ca_ref, sb_ref, cb_ref, out_ref):
    p = pos_ref[...]
    a = jnp.floor(p * (1.0 / SPLIT))
    b = p - a * SPLIT
    ia = lax.broadcasted_iota(jnp.int32, (blk, NA), 1).astype(jnp.float32)
    ib = lax.broadcasted_iota(jnp.int32, (blk, NB), 1).astype(jnp.float32)
    oh_a = (a == ia).astype(jnp.float32)
    oh_b = (b == ib).astype(jnp.float32)
    sA = jnp.dot(oh_a, sa_ref[...], preferred_element_type=jnp.float32)
    cA = jnp.dot(oh_a, ca_ref[...], preferred_element_type=jnp.float32)
    sB = jnp.dot(oh_b, sb_ref[...], preferred_element_type=jnp.float32)
    cB = jnp.dot(oh_b, cb_ref[...], preferred_element_type=jnp.float32)
    sin_out = sA * cB + cA * sB
    cos_out = cA * cB - sA * sB
    nonpad = (p != float(PAD)).astype(jnp.float32)
    out_ref[...] = jnp.concatenate([sin_out, cos_out], axis=1) * nonpad

  tbl = lambda: pl.BlockSpec((NA, half), lambda i: (0, 0))
  tblb = lambda: pl.BlockSpec((NB, half), lambda i: (0, 0))
  return pl.pallas_call(
      body,
      grid=(n_tok // blk,),
      in_specs=[
          pl.BlockSpec((blk, 1), lambda i: (i, 0)),
          tbl(), tbl(), tblb(), tblb(),
      ],
      out_specs=pl.BlockSpec((blk, dim), lambda i: (i, 0)),
      out_shape=jax.ShapeDtypeStruct((n_tok, dim), jnp.float32),
  )


def kernel(input, weights):
  bsz, seq_len = input.shape
  dim = weights.shape[1]
  n_tok = bsz * seq_len
  half = dim // 2

  pos = _build_positions_sc(n_tok)(input.reshape(-1))
  pos_f = pos.astype(jnp.float32).reshape(n_tok, 1)
  scale = math.log(10000.0) / (half - 1)
  freq = jnp.exp(jnp.arange(half, dtype=jnp.float32) * -scale)
  ang_a = (jnp.arange(NA, dtype=jnp.float32) * SPLIT)[:, None] * freq[None, :]
  ang_b = jnp.arange(NB, dtype=jnp.float32)[:, None] * freq[None, :]
  sa, ca = jnp.sin(ang_a), jnp.cos(ang_a)
  sb, cb = jnp.sin(ang_b), jnp.cos(ang_b)

  out = _build_rows_tc(n_tok, dim, 256)(pos_f, sa, ca, sb, cb)
  return out.reshape(bsz, seq_len, dim)
